# Initial kernel scaffold; baseline (speedup 1.0000x reference)
import jax
import jax.numpy as jnp
from jax.experimental import pallas as pl


def kernel(nodefeature, adjm, params):
    raise NotImplementedError("write your pallas kernel here")


# speedup vs baseline: 7.1749x; 7.1749x over previous
import functools
import jax
import jax.numpy as jnp
from jax import lax
from jax.experimental import pallas as pl
from jax.experimental.pallas import tpu as pltpu
from jax.experimental.pallas import tpu_sc as plsc

_N = 512
_H = 8
_DH = 16
_D = 128
_L = 2
_TOPK = 16
_F32 = jnp.float32

_info = plsc.get_sparse_core_info()
_NC, _NS = _info.num_cores, _info.num_subcores
_NW = _NC * _NS
_RPW = _H * _N // _NW


def _sigmoid(x):
    return 1.0 / (1.0 + jnp.exp(-x))


def _emb_kernel(nf, embWT, embb, out_ref):
    out_ref[:] = jnp.dot(nf[:], embWT[:], preferred_element_type=_F32) + embb[:]


def _scores_kernel(h_in, adjmT, ew, eb, WqT, bq, WkT, bk, WvT, bv, We, be,
                   pt_ref, v_ref):
    h = h_in[:]
    q = jnp.dot(h, WqT[:], preferred_element_type=_F32) + bq[:]
    k = jnp.dot(h, WkT[:], preferred_element_type=_F32) + bk[:]
    v_ref[:] = jnp.dot(h, WvT[:], preferred_element_type=_F32) + bv[:]
    adjbT = (adjmT[:] > 0).astype(_F32)
    Wel = We[:]
    c1 = jnp.sum(Wel * ew[:], axis=1, keepdims=True)
    c2 = jnp.sum(Wel * eb[:], axis=1, keepdims=True)
    bel = be[:]
    for hh in range(_H):
        qh = q[:, hh * _DH:(hh + 1) * _DH]
        kh = k[:, hh * _DH:(hh + 1) * _DH]
        scT = jax.lax.dot_general(
            kh, qh, (((1,), (1,)), ((), ())),
            preferred_element_type=_F32) * 0.25
        scT = scT + adjbT * c1[hh:hh + 1] + (c2[hh:hh + 1] + bel[hh:hh + 1])
        cm = jnp.max(scT, axis=0, keepdims=True)
        e = jnp.exp(scT - cm)
        for ib in range(_D // 32):
            pt_ref[hh * 4 + ib] = e[:, ib * 128:(ib + 1) * 128]


def _sc_thr_kernel(pt_hbm, out_hbm, vm, thrv):
    wid = lax.axis_index("s") * _NC + lax.axis_index("c")
    pltpu.sync_copy(pt_hbm.at[wid], vm)
    for sb in range(8):
        c0 = sb * 16

        def _knock(kk, m):
            def _scan(jb, acc):
                base = jb * 16 * 128 + c0
                for r in range(16):
                    x = vm[pl.ds(base + r * 128, 16)]
                    acc = jnp.maximum(acc, jnp.where(x < m, x, 0.0))
                return acc
            return lax.fori_loop(0, _N // 16, _scan,
                                 jnp.zeros((16,), _F32))

        m = lax.fori_loop(0, _TOPK - 1, _knock, jnp.ones((16,), _F32))
        thrv[pl.ds(c0, 16)] = m
    pltpu.sync_copy(thrv, out_hbm.at[pl.ds(wid * _RPW, _RPW)])


_sc_thr = pl.kernel(
    _sc_thr_kernel,
    mesh=plsc.VectorSubcoreMesh(core_axis_name="c", subcore_axis_name="s"),
    out_type=jax.ShapeDtypeStruct((_H * _N,), _F32),
    scratch_types=[
        pltpu.VMEM((_N * _D,), _F32),
        pltpu.VMEM((_RPW,), _F32),
    ],
)


def _tail_kernel(h_in, h0_in, pt, thr2, v_in, WoT, bo, ln1w, ln1b, ln2w,
                 ln2b, gng, gnb, W1T, b1, W2T, b2, WihT, bih_t,
                 out_ref, HA):
    h = h_in[:]
    v = v_in[:]
    for hh in range(_H):
        vh = v[:, hh * _DH:(hh + 1) * _DH]
        for ib in range(4):
            blk = pt[hh * 4 + ib]
            th = thr2[hh:hh + 1, ib * 128:(ib + 1) * 128]
            masked = jnp.where(blk >= th, blk, 0.0)
            denom = jnp.sum(masked, axis=0, keepdims=True)
            attnT = masked / denom
            hab = jax.lax.dot_general(
                attnT, vh, (((0,), (0,)), ((), ())),
                preferred_element_type=_F32)
            HA[ib * 128:(ib + 1) * 128, hh * _DH:(hh + 1) * _DH] = hab
    ha = jnp.dot(HA[:], WoT[:], preferred_element_type=_F32) + bo[:]
    h = h + ha
    mu = jnp.mean(h, axis=1, keepdims=True)
    xc = h - mu
    var = jnp.mean(xc * xc, axis=1, keepdims=True)
    h = ln1w[:] * xc / jnp.sqrt(var + 1e-5) + ln1b[:]

    gates = jnp.dot(h, WihT[:], preferred_element_type=_F32) + bih_t[:]
    ig = gates[:, 0:_D]
    gg = gates[:, 2 * _D:3 * _D]
    og = gates[:, 3 * _D:4 * _D]
    c_next = _sigmoid(ig) * jnp.tanh(gg)
    h = h + _sigmoid(og) * jnp.tanh(c_next)

    mu0 = jnp.mean(h, axis=0, keepdims=True)
    xc0 = h - mu0
    var0 = jnp.mean(xc0 * xc0, axis=0, keepdims=True)
    h = gng[:] * xc0 / jnp.sqrt(var0 + 1e-5) + gnb[:]

    t = jnp.maximum(jnp.dot(h, W1T[:], preferred_element_type=_F32) + b1[:],
                    0.0)
    f = jnp.dot(t, W2T[:], preferred_element_type=_F32) + b2[:]
    hf = h + f
    mu2 = jnp.mean(hf, axis=1, keepdims=True)
    xc2 = hf - mu2
    var2 = jnp.mean(xc2 * xc2, axis=1, keepdims=True)
    h = ln2w[:] * xc2 / jnp.sqrt(var2 + 1e-5) + ln2b[:]
    out_ref[:] = h + h0_in[:]


def _out_kernel(h_in, outWT, outb, out_ref):
    out_ref[:] = jnp.dot(h_in[:], outWT[:],
                         preferred_element_type=_F32) + outb[:]


def kernel(nodefeature, adjm, params):
    p = params
    t2 = lambda w: w.swapaxes(-1, -2)
    r2 = lambda b: b.reshape(_L, 1, -1)

    h = pl.pallas_call(
        _emb_kernel,
        out_shape=jax.ShapeDtypeStruct((_N, _D), _F32),
    )(nodefeature, p['emb_W'].T, p['emb_b'].reshape(1, _D))
    h0 = h

    adjmT = adjm.T
    ew = p['edge_W'][:, 0].reshape(1, _D)
    eb = p['edge_b'].reshape(1, _D)
    WqT = t2(p['Wq']); WkT = t2(p['Wk']); WvT = t2(p['Wv']); WoT = t2(p['Wo'])
    W1T = t2(p['ffn_W1']); W2T = t2(p['ffn_W2']); WihT = t2(p['lstm_Wih'])
    bq = r2(p['bq']); bk = r2(p['bk']); bv = r2(p['bv']); bo = r2(p['bo'])
    b1 = r2(p['ffn_b1']); b2 = r2(p['ffn_b2'])
    bih_t = r2(p['lstm_bih'] + p['lstm_bhh'])
    be = p['be'].reshape(_L, _H, 1)

    for l in range(_L):
        pt, v = pl.pallas_call(
            _scores_kernel,
            out_shape=[
                jax.ShapeDtypeStruct((4 * _H, _N, 128), _F32),
                jax.ShapeDtypeStruct((_N, _D), _F32),
            ],
        )(h, adjmT, ew, eb, WqT[l], bq[l], WkT[l], bk[l], WvT[l], bv[l],
          p['We'][l], be[l])

        thr = _sc_thr(pt.reshape(4 * _H, _N * 128))
        thr2 = thr.reshape(_H, _N)

        h = pl.pallas_call(
            _tail_kernel,
            out_shape=jax.ShapeDtypeStruct((_N, _D), _F32),
            scratch_shapes=[pltpu.VMEM((_N, _D), _F32)],
        )(h, h0, pt, thr2, v, WoT[l], bo[l],
          r2(p['ln1_w'])[l], r2(p['ln1_b'])[l],
          r2(p['ln2_w'])[l], r2(p['ln2_b'])[l],
          r2(p['gn_g'])[l], r2(p['gn_b'])[l],
          W1T[l], b1[l], W2T[l], b2[l], WihT[l], bih_t[l])

    return pl.pallas_call(
        _out_kernel,
        out_shape=jax.ShapeDtypeStruct((_N, _D), _F32),
    )(h, p['out_W'].T, p['out_b'].reshape(1, _D))


# speedup vs baseline: 22.2258x; 3.0977x over previous
import jax
import jax.numpy as jnp
from jax.experimental import pallas as pl
from jax.experimental.pallas import tpu as pltpu

_N = 512
_H = 8
_DH = 16
_D = 128
_L = 2
_TOPK = 16
_F32 = jnp.float32


def _sigmoid(x):
    return 1.0 / (1.0 + jnp.exp(-x))


def _fwd_kernel(
    nf, adjm, embWT, embb, ew, eb,
    WqT, bq, WkT, bk, WvT, bv, We, be, WoT, bo,
    ln1w, ln1b, ln2w, ln2b, gng, gnb,
    W1T, b1, W2T, b2, WihT, bih_t,
    outWT, outb,
    out_ref, SW, HA,
):
    h = jnp.dot(nf[:], embWT[:], preferred_element_type=_F32) + embb[:]
    h0 = h
    adjb = (adjm[:] > 0).astype(_F32)

    for l in range(_L):
        resid = h
        q = jnp.dot(h, WqT[l], preferred_element_type=_F32) + bq[l]
        k = jnp.dot(h, WkT[l], preferred_element_type=_F32) + bk[l]
        v = jnp.dot(h, WvT[l], preferred_element_type=_F32) + bv[l]
        Wel = We[l]
        c1 = jnp.sum(Wel * ew[:], axis=1, keepdims=True)
        c2 = jnp.sum(Wel * eb[:], axis=1, keepdims=True)
        bel = be[l]

        for hh in range(_H):
            qh = q[:, hh * _DH:(hh + 1) * _DH]
            kh = k[:, hh * _DH:(hh + 1) * _DH]
            qk = jax.lax.dot_general(
                qh, kh, (((1,), (1,)), ((), ())),
                preferred_element_type=_F32) * 0.25
            sc = qk + adjb * c1[hh:hh + 1] + (c2[hh:hh + 1] + bel[hh:hh + 1])
            rowmax = jnp.max(sc, axis=1, keepdims=True)
            SW[:, hh * _N:(hh + 1) * _N] = jnp.exp(sc - rowmax)

        ones = jnp.ones((_N, 1), _F32)
        thrs = [ones] * _H
        for _p in range(3):
            for hh in range(_H):
                sl = slice(hh * _N, (hh + 1) * _N)
                cur = SW[:, sl]
                m = thrs[hh]
                for _ in range(5):
                    m = jnp.max(jnp.where(cur < m, cur, 0.0),
                                axis=1, keepdims=True)
                thrs[hh] = m

        for hh in range(_H):
            sl = slice(hh * _N, (hh + 1) * _N)
            cur = SW[:, sl]
            thr = thrs[hh]
            p_un = jnp.where(cur >= thr, cur, 0.0)
            denom = jnp.sum(p_un, axis=1, keepdims=True)
            attn = p_un / denom
            vh = v[:, hh * _DH:(hh + 1) * _DH]
            HA[:, hh * _DH:(hh + 1) * _DH] = jnp.dot(
                attn, vh, preferred_element_type=_F32)

        ha = jnp.dot(HA[:], WoT[l], preferred_element_type=_F32) + bo[l]
        h = resid + ha
        mu = jnp.mean(h, axis=1, keepdims=True)
        xc = h - mu
        var = jnp.mean(xc * xc, axis=1, keepdims=True)
        h = ln1w[l] * xc / jnp.sqrt(var + 1e-5) + ln1b[l]

        gates = jnp.dot(h, WihT[l], preferred_element_type=_F32) + bih_t[l]
        ig = gates[:, 0:_D]
        gg = gates[:, 2 * _D:3 * _D]
        og = gates[:, 3 * _D:4 * _D]
        c_next = _sigmoid(ig) * jnp.tanh(gg)
        h = h + _sigmoid(og) * jnp.tanh(c_next)

        mu0 = jnp.mean(h, axis=0, keepdims=True)
        xc0 = h - mu0
        var0 = jnp.mean(xc0 * xc0, axis=0, keepdims=True)
        h = gng[l] * xc0 / jnp.sqrt(var0 + 1e-5) + gnb[l]

        t = jnp.maximum(
            jnp.dot(h, W1T[l], preferred_element_type=_F32) + b1[l], 0.0)
        f = jnp.dot(t, W2T[l], preferred_element_type=_F32) + b2[l]
        hf = h + f
        mu2 = jnp.mean(hf, axis=1, keepdims=True)
        xc2 = hf - mu2
        var2 = jnp.mean(xc2 * xc2, axis=1, keepdims=True)
        h = ln2w[l] * xc2 / jnp.sqrt(var2 + 1e-5) + ln2b[l]

        h = h + h0

    out_ref[:] = jnp.dot(h, outWT[:], preferred_element_type=_F32) + outb[:]


def kernel(nodefeature, adjm, params):
    p = params
    tL = lambda w: jnp.swapaxes(w, 1, 2)
    r1 = lambda b: b.reshape(_L, 1, -1)
    args = (
        nodefeature, adjm,
        p['emb_W'].T, p['emb_b'].reshape(1, _D),
        p['edge_W'][:, 0].reshape(1, _D), p['edge_b'].reshape(1, _D),
        tL(p['Wq']), r1(p['bq']), tL(p['Wk']), r1(p['bk']),
        tL(p['Wv']), r1(p['bv']), p['We'], p['be'].reshape(_L, _H, 1),
        tL(p['Wo']), r1(p['bo']),
        r1(p['ln1_w']), r1(p['ln1_b']), r1(p['ln2_w']), r1(p['ln2_b']),
        r1(p['gn_g']), r1(p['gn_b']),
        tL(p['ffn_W1']), r1(p['ffn_b1']), tL(p['ffn_W2']), r1(p['ffn_b2']),
        tL(p['lstm_Wih']), r1(p['lstm_bih'] + p['lstm_bhh']),
        p['out_W'].T, p['out_b'].reshape(1, _D),
    )
    return pl.pallas_call(
        _fwd_kernel,
        out_shape=jax.ShapeDtypeStruct((_N, _D), _F32),
        scratch_shapes=[
            pltpu.VMEM((_N, _H * _N), _F32),
            pltpu.VMEM((_N, _D), _F32),
        ],
        interpret=False,
    )(*args)
